# baseline (speedup 1.0000x reference)
import jax
import jax.numpy as jnp
from jax.experimental import pallas as pl
from jax.experimental.pallas import tpu as pltpu

_Q = 1024
_D = 768
_K = 100000
_BK = 5000
_NB = _K // _BK
_HW = 2560
_STARTS = (0, _BK - _HW)
_NEG = -3.4e38
_IMAX = 2147483647


def _fold(s, base0, R, T):
    for c in range(_HW // 128):
        sc = jax.lax.slice_in_dim(s, c * 128, (c + 1) * 128, axis=1)
        upd = sc > R
        R = jnp.where(upd, sc, R)
        T = jnp.where(upd, jnp.int32(base0 + c * 128), T)
    return R, T


def _topk_kernel(q_ref, k_ref, val_ref, idx_ref, R_ref, T_ref):
    j = pl.program_id(0)

    @pl.when(j == 0)
    def _init():
        R_ref[...] = jnp.full((_Q, 128), _NEG, jnp.float32)
        T_ref[...] = jnp.zeros((_Q, 128), jnp.int32)

    R = R_ref[...]
    T = T_ref[...]
    for start in _STARTS:
        kh = k_ref[start:start + _HW, :]
        s = jax.lax.dot_general(
            q_ref[...], kh,
            dimension_numbers=(((1,), (1,)), ((), ())),
            preferred_element_type=jnp.float32,
        )
        R, T = _fold(s, j * _BK + start, R, T)
    R_ref[...] = R
    T_ref[...] = T

    @pl.when(j == _NB - 1)
    def _extract():
        Rf = R_ref[...]
        v = jnp.max(Rf, axis=1, keepdims=True)
        lane = jax.lax.broadcasted_iota(jnp.int32, (_Q, 128), 1)
        gidx = T_ref[...] + lane
        idxv = jnp.min(jnp.where(Rf == v, gidx, _IMAX), axis=1, keepdims=True)
        val_ref[...] = v
        idx_ref[...] = idxv


def kernel(queries, keys):
    top_vals, top_idx = pl.pallas_call(
        _topk_kernel,
        grid=(_NB,),
        in_specs=[
            pl.BlockSpec((_Q, _D), lambda j: (0, 0)),
            pl.BlockSpec((_BK, _D), lambda j: (j, 0)),
        ],
        out_specs=[
            pl.BlockSpec((_Q, 1), lambda j: (0, 0)),
            pl.BlockSpec((_Q, 1), lambda j: (0, 0)),
        ],
        out_shape=[
            jax.ShapeDtypeStruct((_Q, 1), jnp.float32),
            jax.ShapeDtypeStruct((_Q, 1), jnp.int32),
        ],
        scratch_shapes=[
            pltpu.VMEM((_Q, 128), jnp.float32),
            pltpu.VMEM((_Q, 128), jnp.int32),
        ],
        compiler_params=pltpu.CompilerParams(
            dimension_semantics=("arbitrary",),
        ),
    )(queries, keys)
    return top_vals, top_idx
